# baseline (speedup 1.0000x reference)
import functools

import jax
import jax.numpy as jnp
from jax.experimental import pallas as pl

B = 16
D = 64
K = 4
C_BLK = 4096


def _body(codes_ref, cents_ref, out_ref):
    codes = codes_ref[...]
    ones = jnp.ones((1, D), jnp.float32)
    dmin = None
    for k in range(K):
        ck = cents_ref[:, k * D:(k + 1) * D]
        sim = jax.lax.dot_general(
            codes, ck, (((1,), (1,)), ((), ())),
            preferred_element_type=jnp.float32)
        n2 = jax.lax.dot_general(
            ones, ck * ck, (((1,), (1,)), ((), ())),
            preferred_element_type=jnp.float32)
        inv = 1.0 / jnp.maximum(jnp.sqrt(n2), 1e-12)
        d = 1.0 - sim * inv
        dmin = d if dmin is None else jnp.minimum(dmin, d)
    out_ref[...] = dmin


@jax.jit
def kernel(codes, centroids):
    c = centroids.shape[0]
    cents2d = centroids.reshape(c, K * D)
    grid = (c + C_BLK - 1) // C_BLK
    return pl.pallas_call(
        _body,
        grid=(grid,),
        in_specs=[
            pl.BlockSpec((B, D), lambda i: (0, 0)),
            pl.BlockSpec((C_BLK, K * D), lambda i: (i, 0)),
        ],
        out_specs=pl.BlockSpec((B, C_BLK), lambda i: (0, i)),
        out_shape=jax.ShapeDtypeStruct((B, c), jnp.float32),
    )(codes, cents2d)
